# baseline (speedup 1.0000x reference)
import functools

import jax
import jax.numpy as jnp
from jax import lax
from jax.experimental import pallas as pl
from jax.experimental.pallas import tpu as pltpu
from jax.experimental.pallas import tpu_sc as plsc

NC = 2
NS = 16
NW = NC * NS
F0 = 0.1
C = 64
ZC = 64


def _sc_degrees(np_, nch, nch0, nch1, idx4, zeros_np, ones16_arr):
    mesh = plsc.VectorSubcoreMesh(core_axis_name="c", subcore_axis_name="s")

    @functools.partial(
        pl.kernel,
        out_type=(
            jax.ShapeDtypeStruct((NW * np_,), jnp.float32),
            jax.ShapeDtypeStruct((NW * np_,), jnp.float32),
        ),
        mesh=mesh,
        scratch_types=[
            pltpu.VMEM((nch, 2, C), jnp.int32),
            pltpu.VMEM((np_,), jnp.float32),
            pltpu.VMEM((np_,), jnp.float32),
            pltpu.VMEM((16,), jnp.float32),
        ],
        compiler_params=pltpu.CompilerParams(needs_layout_passes=False),
    )
    def deg_kernel(idx_hbm, zeros_hbm, ones_hbm, dego_hbm, degi_hbm,
                   idxv, dego_loc, degi_loc, onesv):
        cid = lax.axis_index("c")
        sid = lax.axis_index("s")
        wid = cid * NS + sid
        pltpu.sync_copy(zeros_hbm, dego_loc)
        pltpu.sync_copy(zeros_hbm, degi_loc)
        pltpu.sync_copy(idx_hbm.at[wid], idxv)
        pltpu.sync_copy(ones_hbm, onesv)
        ones16 = onesv[...]
        ncw = jnp.where(cid == 0, nch0, nch1)

        def body(c, carry):
            src_row = idxv.at[c, 0]
            dst_row = idxv.at[c, 1]
            for k in range(C // 16):
                s16 = src_row[pl.ds(k * 16, 16)]
                d16 = dst_row[pl.ds(k * 16, 16)]
                plsc.addupdate_scatter(dego_loc, [s16], ones16)
                plsc.addupdate_scatter(degi_loc, [d16], ones16)
            return carry

        lax.fori_loop(0, ncw, body, 0)
        osl = pl.ds(wid * np_, np_)
        pltpu.sync_copy(dego_loc, dego_hbm.at[osl])
        pltpu.sync_copy(degi_loc, degi_hbm.at[osl])

    dego, degi = deg_kernel(idx4, zeros_np, ones16_arr)
    return dego.reshape(NW, np_), degi.reshape(NW, np_)


def _sc_spmm(np_, nch, nch0, nch1, xs, norm_in, idx4, zeros_np, zeros2):
    d = xs.shape[1]
    rpt = np_ // NS
    nzc = rpt // ZC

    mesh = plsc.VectorSubcoreMesh(core_axis_name="c", subcore_axis_name="s")

    @functools.partial(
        pl.kernel,
        out_type=(
            jax.ShapeDtypeStruct((NC, np_, d), jnp.float32),
            jax.ShapeDtypeStruct((NW * np_,), jnp.float32),
        ),
        mesh=mesh,
        scratch_types=[
            pltpu.VMEM((2, C), jnp.int32),
            pltpu.VMEM((2, C), jnp.int32),
            pltpu.VMEM((2, C), jnp.int32),
            pltpu.VMEM((2, C), jnp.int32),
            pltpu.VMEM((C, d), jnp.float32),
            pltpu.VMEM((C, d), jnp.float32),
            pltpu.VMEM((np_,), jnp.float32),
            pltpu.VMEM((np_,), jnp.float32),
            pltpu.VMEM_SHARED((np_, d), jnp.float32),
            pltpu.SemaphoreType.DMA,
            pltpu.SemaphoreType.DMA,
            pltpu.SemaphoreType.DMA,
            pltpu.SemaphoreType.DMA,
            pltpu.SemaphoreType.DMA,
            pltpu.SemaphoreType.DMA,
            pltpu.SemaphoreType.DMA,
            pltpu.SemaphoreType.DMA,
        ],
        compiler_params=pltpu.CompilerParams(needs_layout_passes=False),
    )
    def spmm_kernel(xs_hbm, nin_hbm, idx_hbm, zeros_np_hbm, zeros2_hbm,
                    agg_hbm, w_hbm, idx0, idx1, idx2, idx3,
                    rows_a, rows_b, nin_loc, w_loc, agg_sh,
                    sga, sgb, ssa, ssb, si0, si1, si2, si3):
        cid = lax.axis_index("c")
        sid = lax.axis_index("s")
        wid = cid * NS + sid
        ncw = jnp.where(cid == 0, nch0, nch1)
        pltpu.sync_copy(zeros2_hbm, rows_a.at[pl.ds(0, ZC)])
        for k in range(nzc):
            pltpu.sync_copy(rows_a.at[pl.ds(0, ZC)],
                            agg_sh.at[pl.ds(sid * rpt + k * ZC, ZC)])
        pltpu.sync_copy(nin_hbm, nin_loc)
        pltpu.sync_copy(zeros_np_hbm, w_loc)
        plsc.subcore_barrier()

        pltpu.async_copy(idx_hbm.at[wid, 0], idx0, si0)
        pltpu.async_copy(idx_hbm.at[wid, 1], idx1, si1)

        def scalar_w(idx):
            for k in range(C // 16):
                s16 = idx[0, pl.ds(k * 16, 16)]
                d16 = idx[1, pl.ds(k * 16, 16)]
                vals = plsc.load_gather(nin_loc, [d16])
                plsc.addupdate_scatter(w_loc, [s16], vals)

        def body(g, carry):
            c0 = 4 * g
            @pl.when(g > 0)
            def _():
                pltpu.make_async_copy(rows_a, agg_sh.at[idx2.at[1]], ssa).wait()
                pltpu.make_async_copy(rows_b, agg_sh.at[idx3.at[1]], ssb).wait()

            pltpu.async_copy(idx_hbm.at[wid, c0 + 2], idx2, si2)
            pltpu.async_copy(idx_hbm.at[wid, c0 + 3], idx3, si3)

            pltpu.make_async_copy(idx_hbm.at[wid, c0], idx0, si0).wait()
            g0 = pltpu.async_copy(xs_hbm.at[idx0.at[0]], rows_a, sga)
            pltpu.make_async_copy(idx_hbm.at[wid, c0 + 1], idx1, si1).wait()
            g1 = pltpu.async_copy(xs_hbm.at[idx1.at[0]], rows_b, sgb)

            g0.wait()
            s0 = pltpu.async_copy(rows_a, agg_sh.at[idx0.at[1]], ssa, add=True)
            scalar_w(idx0)
            g1.wait()
            s1 = pltpu.async_copy(rows_b, agg_sh.at[idx1.at[1]], ssb, add=True)
            scalar_w(idx1)

            nx0 = jnp.minimum(c0 + 4, ncw - 2)
            nx1 = jnp.minimum(c0 + 5, ncw - 1)
            s0.wait()
            pltpu.async_copy(idx_hbm.at[wid, nx0], idx0, si0)
            pltpu.make_async_copy(idx_hbm.at[wid, c0 + 2], idx2, si2).wait()
            g2 = pltpu.async_copy(xs_hbm.at[idx2.at[0]], rows_a, sga)
            s1.wait()
            pltpu.async_copy(idx_hbm.at[wid, nx1], idx1, si1)
            pltpu.make_async_copy(idx_hbm.at[wid, c0 + 3], idx3, si3).wait()
            g3 = pltpu.async_copy(xs_hbm.at[idx3.at[0]], rows_b, sgb)

            g2.wait()
            pltpu.async_copy(rows_a, agg_sh.at[idx2.at[1]], ssa, add=True)
            scalar_w(idx2)
            g3.wait()
            pltpu.async_copy(rows_b, agg_sh.at[idx3.at[1]], ssb, add=True)
            scalar_w(idx3)
            return carry

        lax.fori_loop(0, ncw // 4, body, 0)
        pltpu.make_async_copy(rows_a, agg_sh.at[idx2.at[1]], ssa).wait()
        pltpu.make_async_copy(rows_b, agg_sh.at[idx3.at[1]], ssb).wait()
        pltpu.make_async_copy(idx_hbm.at[wid, ncw - 2], idx0, si0).wait()
        pltpu.make_async_copy(idx_hbm.at[wid, ncw - 1], idx1, si1).wait()
        plsc.subcore_barrier()
        for k in range(nzc):
            csl = pl.ds(sid * rpt + k * ZC, ZC)
            pltpu.sync_copy(agg_sh.at[csl], rows_a.at[pl.ds(0, ZC)])
            pltpu.sync_copy(rows_a.at[pl.ds(0, ZC)], agg_hbm.at[cid, csl])
        pltpu.sync_copy(w_loc, w_hbm.at[pl.ds(wid * np_, np_)])

    agg_p, w_p = spmm_kernel(xs, norm_in, idx4, zeros_np, zeros2)
    return agg_p, w_p.reshape(NW, np_)


def _tc_norms(dop_t, dip_t, x_pad):
    np_, d = x_pad.shape

    def body(dop_ref, dip_ref, x_ref, xs_ref, no_ref, ni_ref):
        dego = jnp.sum(dop_ref[...], axis=1, keepdims=True)
        degi = jnp.sum(dip_ref[...], axis=1, keepdims=True)
        no = lax.rsqrt(jnp.maximum(dego, 1.0))
        ni = lax.rsqrt(jnp.maximum(degi, 1.0))
        no_ref[...] = no
        ni_ref[...] = ni
        xs_ref[...] = x_ref[...] * no

    return pl.pallas_call(
        body,
        out_shape=(
            jax.ShapeDtypeStruct((np_, d), jnp.float32),
            jax.ShapeDtypeStruct((np_, 1), jnp.float32),
            jax.ShapeDtypeStruct((np_, 1), jnp.float32),
        ),
    )(dop_t, dip_t, x_pad)


def _tc_final(n_real, agg0, agg1, w_p, ni, no, W1, b1, W2, b2, Wl, bl):
    np_, d = agg0.shape

    def body(agg0_ref, agg1_ref, wp_ref, ni_ref, no_ref,
             W1_ref, b1_ref, W2_ref, b2_ref, Wl_ref, bl_ref, out_ref):
        t = (agg0_ref[...] + agg1_ref[...]) * ni_ref[...]
        h1 = jnp.dot(t, W1_ref[...], preferred_element_type=jnp.float32)
        h1 = jnp.maximum(h1 + b1_ref[...], 0.0)
        w = jnp.sum(wp_ref[...], axis=1, keepdims=True) * no_ref[...]
        mask = lax.broadcasted_iota(jnp.int32, (np_, 1), 0) < n_real
        w = jnp.where(mask, w, 0.0)
        v = jnp.sum(h1 * w, axis=0, keepdims=True)
        readout = jnp.dot(v, W2_ref[...], preferred_element_type=jnp.float32)
        readout = readout * (1.0 / n_real) + b2_ref[...]
        out = jnp.dot(readout, Wl_ref[...], preferred_element_type=jnp.float32)
        out_ref[...] = out + bl_ref[...]

    return pl.pallas_call(
        body,
        out_shape=jax.ShapeDtypeStruct((1, bl.shape[-1]), jnp.float32),
    )(agg0, agg1, w_p, ni, no, W1, b1, W2, b2, Wl, bl)


def kernel(x, edge_index, W1, b1, W2, b2, Wl, bl):
    n, d = x.shape
    e = edge_index.shape[1]

    np_ = ((n + NS * 128 - 1) // (NS * 128)) * (NS * 128)
    pad_node = np_ - 1

    def _rup4(ne):
        return max(4, 4 * ((ne + 4 * NS * C - 1) // (4 * NS * C)))

    e0 = int(e * F0)
    nch0 = _rup4(e0)
    e0 = min(e, nch0 * C * NS)
    nch1 = _rup4(e - e0)
    nch = max(nch0, nch1)

    src = edge_index[0].astype(jnp.int32)
    dst = edge_index[1].astype(jnp.int32)

    def _layout(v):
        fill = jnp.full(((nch0 + nch1) * C * NS - e,), pad_node, jnp.int32)
        vp = jnp.concatenate([v, fill])
        v0 = vp[:nch0 * C * NS].reshape(NS, nch0, C)
        v1 = vp[nch0 * C * NS:].reshape(NS, nch1, C)
        if nch0 < nch:
            v0 = jnp.concatenate(
                [v0, jnp.full((NS, nch - nch0, C), pad_node, jnp.int32)], 1)
        if nch1 < nch:
            v1 = jnp.concatenate(
                [v1, jnp.full((NS, nch - nch1, C), pad_node, jnp.int32)], 1)
        return jnp.concatenate([v0, v1], 0)

    src3 = _layout(src)
    dst3 = _layout(dst)
    idx4 = jnp.stack([src3, dst3], axis=2)

    x_pad = jnp.zeros((np_, d), jnp.float32).at[:n].set(x)

    zeros_np = jnp.zeros((np_,), jnp.float32)
    zeros2 = jnp.zeros((ZC, d), jnp.float32)

    deg_out_p, deg_in_p = _sc_degrees(np_, nch, nch0, nch1, idx4, zeros_np,
                                      jnp.ones((16,), jnp.float32))

    xs, norm_out, norm_in = _tc_norms(deg_out_p.T, deg_in_p.T, x_pad)

    agg_p, w_p = _sc_spmm(np_, nch, nch0, nch1, xs, norm_in.reshape(np_),
                          idx4, zeros_np, zeros2)

    out = _tc_final(n, agg_p[0], agg_p[1], w_p.T,
                    norm_in, norm_out, W1, b1[None, :], W2, b2[None, :],
                    Wl, bl[None, :])
    return out


# speedup vs baseline: 1.1529x; 1.1529x over previous
import functools

import jax
import jax.numpy as jnp
from jax import lax
from jax.experimental import pallas as pl
from jax.experimental.pallas import tpu as pltpu
from jax.experimental.pallas import tpu_sc as plsc

NC = 2
NS = 16
NW = NC * NS
F0 = 0.15
C = 64
ZC = 64


def _sc_degrees(np_, nch, nch0, nch1, idx4, zeros_np, ones16_arr):
    mesh = plsc.VectorSubcoreMesh(core_axis_name="c", subcore_axis_name="s")

    @functools.partial(
        pl.kernel,
        out_type=(
            jax.ShapeDtypeStruct((NW * np_,), jnp.float32),
            jax.ShapeDtypeStruct((NW * np_,), jnp.float32),
        ),
        mesh=mesh,
        scratch_types=[
            pltpu.VMEM((nch, 2, C), jnp.int32),
            pltpu.VMEM((np_,), jnp.float32),
            pltpu.VMEM((np_,), jnp.float32),
            pltpu.VMEM((16,), jnp.float32),
        ],
        compiler_params=pltpu.CompilerParams(needs_layout_passes=False),
    )
    def deg_kernel(idx_hbm, zeros_hbm, ones_hbm, dego_hbm, degi_hbm,
                   idxv, dego_loc, degi_loc, onesv):
        cid = lax.axis_index("c")
        sid = lax.axis_index("s")
        wid = cid * NS + sid
        pltpu.sync_copy(zeros_hbm, dego_loc)
        pltpu.sync_copy(zeros_hbm, degi_loc)
        pltpu.sync_copy(idx_hbm.at[wid], idxv)
        pltpu.sync_copy(ones_hbm, onesv)
        ones16 = onesv[...]
        ncw = jnp.where(cid == 0, nch0, nch1)

        def body(c, carry):
            src_row = idxv.at[c, 0]
            dst_row = idxv.at[c, 1]
            for k in range(C // 16):
                s16 = src_row[pl.ds(k * 16, 16)]
                d16 = dst_row[pl.ds(k * 16, 16)]
                plsc.addupdate_scatter(dego_loc, [s16], ones16)
                plsc.addupdate_scatter(degi_loc, [d16], ones16)
            return carry

        lax.fori_loop(0, ncw, body, 0)
        osl = pl.ds(wid * np_, np_)
        pltpu.sync_copy(dego_loc, dego_hbm.at[osl])
        pltpu.sync_copy(degi_loc, degi_hbm.at[osl])

    dego, degi = deg_kernel(idx4, zeros_np, ones16_arr)
    return dego.reshape(NW, np_), degi.reshape(NW, np_)


def _sc_spmm(np_, nch, nch0, nch1, xs, norm_in, idx4, zeros_np, zeros2):
    d = xs.shape[1]
    rpt = np_ // NS
    nzc = rpt // ZC

    mesh = plsc.VectorSubcoreMesh(core_axis_name="c", subcore_axis_name="s")

    @functools.partial(
        pl.kernel,
        out_type=(
            jax.ShapeDtypeStruct((NC, np_, d), jnp.float32),
            jax.ShapeDtypeStruct((NW * np_,), jnp.float32),
        ),
        mesh=mesh,
        scratch_types=[
            pltpu.VMEM((2, C), jnp.int32),
            pltpu.VMEM((2, C), jnp.int32),
            pltpu.VMEM((2, C), jnp.int32),
            pltpu.VMEM((2, C), jnp.int32),
            pltpu.VMEM((C, d), jnp.float32),
            pltpu.VMEM((C, d), jnp.float32),
            pltpu.VMEM((np_,), jnp.float32),
            pltpu.VMEM((np_,), jnp.float32),
            pltpu.VMEM_SHARED((np_, d), jnp.float32),
            pltpu.SemaphoreType.DMA,
            pltpu.SemaphoreType.DMA,
            pltpu.SemaphoreType.DMA,
            pltpu.SemaphoreType.DMA,
            pltpu.SemaphoreType.DMA,
            pltpu.SemaphoreType.DMA,
            pltpu.SemaphoreType.DMA,
            pltpu.SemaphoreType.DMA,
        ],
        compiler_params=pltpu.CompilerParams(needs_layout_passes=False),
    )
    def spmm_kernel(xs_hbm, nin_hbm, idx_hbm, zeros_np_hbm, zeros2_hbm,
                    agg_hbm, w_hbm, idx0, idx1, idx2, idx3,
                    rows_a, rows_b, nin_loc, w_loc, agg_sh,
                    sga, sgb, ssa, ssb, si0, si1, si2, si3):
        cid = lax.axis_index("c")
        sid = lax.axis_index("s")
        wid = cid * NS + sid
        ncw = jnp.where(cid == 0, nch0, nch1)
        pltpu.sync_copy(zeros2_hbm, rows_a.at[pl.ds(0, ZC)])
        for k in range(nzc):
            pltpu.sync_copy(rows_a.at[pl.ds(0, ZC)],
                            agg_sh.at[pl.ds(sid * rpt + k * ZC, ZC)])
        pltpu.sync_copy(nin_hbm, nin_loc)
        pltpu.sync_copy(zeros_np_hbm, w_loc)
        plsc.subcore_barrier()

        pltpu.async_copy(idx_hbm.at[wid, 0], idx0, si0)
        pltpu.async_copy(idx_hbm.at[wid, 1], idx1, si1)

        def scalar_w(idx):
            for k in range(C // 16):
                s16 = idx[0, pl.ds(k * 16, 16)]
                d16 = idx[1, pl.ds(k * 16, 16)]
                vals = plsc.load_gather(nin_loc, [d16])
                plsc.addupdate_scatter(w_loc, [s16], vals)

        def body(g, carry):
            c0 = 4 * g
            @pl.when(g > 0)
            def _():
                pltpu.make_async_copy(rows_a, agg_sh.at[idx2.at[1]], ssa).wait()
                pltpu.make_async_copy(rows_b, agg_sh.at[idx3.at[1]], ssb).wait()

            pltpu.async_copy(idx_hbm.at[wid, c0 + 2], idx2, si2)
            pltpu.async_copy(idx_hbm.at[wid, c0 + 3], idx3, si3)

            pltpu.make_async_copy(idx_hbm.at[wid, c0], idx0, si0).wait()
            g0 = pltpu.async_copy(xs_hbm.at[idx0.at[0]], rows_a, sga)
            pltpu.make_async_copy(idx_hbm.at[wid, c0 + 1], idx1, si1).wait()
            g1 = pltpu.async_copy(xs_hbm.at[idx1.at[0]], rows_b, sgb)

            g0.wait()
            s0 = pltpu.async_copy(rows_a, agg_sh.at[idx0.at[1]], ssa, add=True)
            scalar_w(idx0)
            g1.wait()
            s1 = pltpu.async_copy(rows_b, agg_sh.at[idx1.at[1]], ssb, add=True)
            scalar_w(idx1)

            nx0 = jnp.minimum(c0 + 4, ncw - 2)
            nx1 = jnp.minimum(c0 + 5, ncw - 1)
            s0.wait()
            pltpu.async_copy(idx_hbm.at[wid, nx0], idx0, si0)
            pltpu.make_async_copy(idx_hbm.at[wid, c0 + 2], idx2, si2).wait()
            g2 = pltpu.async_copy(xs_hbm.at[idx2.at[0]], rows_a, sga)
            s1.wait()
            pltpu.async_copy(idx_hbm.at[wid, nx1], idx1, si1)
            pltpu.make_async_copy(idx_hbm.at[wid, c0 + 3], idx3, si3).wait()
            g3 = pltpu.async_copy(xs_hbm.at[idx3.at[0]], rows_b, sgb)

            g2.wait()
            pltpu.async_copy(rows_a, agg_sh.at[idx2.at[1]], ssa, add=True)
            scalar_w(idx2)
            g3.wait()
            pltpu.async_copy(rows_b, agg_sh.at[idx3.at[1]], ssb, add=True)
            scalar_w(idx3)
            return carry

        lax.fori_loop(0, ncw // 4, body, 0)
        pltpu.make_async_copy(rows_a, agg_sh.at[idx2.at[1]], ssa).wait()
        pltpu.make_async_copy(rows_b, agg_sh.at[idx3.at[1]], ssb).wait()
        pltpu.make_async_copy(idx_hbm.at[wid, ncw - 2], idx0, si0).wait()
        pltpu.make_async_copy(idx_hbm.at[wid, ncw - 1], idx1, si1).wait()
        plsc.subcore_barrier()
        for k in range(nzc):
            csl = pl.ds(sid * rpt + k * ZC, ZC)
            pltpu.sync_copy(agg_sh.at[csl], rows_a.at[pl.ds(0, ZC)])
            pltpu.sync_copy(rows_a.at[pl.ds(0, ZC)], agg_hbm.at[cid, csl])
        pltpu.sync_copy(w_loc, w_hbm.at[pl.ds(wid * np_, np_)])

    agg_p, w_p = spmm_kernel(xs, norm_in, idx4, zeros_np, zeros2)
    return agg_p, w_p.reshape(NW, np_)


def _tc_norms(dop_t, dip_t, x_pad):
    np_, d = x_pad.shape

    def body(dop_ref, dip_ref, x_ref, xs_ref, no_ref, ni_ref):
        dego = jnp.sum(dop_ref[...], axis=1, keepdims=True)
        degi = jnp.sum(dip_ref[...], axis=1, keepdims=True)
        no = lax.rsqrt(jnp.maximum(dego, 1.0))
        ni = lax.rsqrt(jnp.maximum(degi, 1.0))
        no_ref[...] = no
        ni_ref[...] = ni
        xs_ref[...] = x_ref[...] * no

    return pl.pallas_call(
        body,
        out_shape=(
            jax.ShapeDtypeStruct((np_, d), jnp.float32),
            jax.ShapeDtypeStruct((np_, 1), jnp.float32),
            jax.ShapeDtypeStruct((np_, 1), jnp.float32),
        ),
    )(dop_t, dip_t, x_pad)


def _tc_final(n_real, agg0, agg1, w_p, ni, no, W1, b1, W2, b2, Wl, bl):
    np_, d = agg0.shape

    def body(agg0_ref, agg1_ref, wp_ref, ni_ref, no_ref,
             W1_ref, b1_ref, W2_ref, b2_ref, Wl_ref, bl_ref, out_ref):
        t = (agg0_ref[...] + agg1_ref[...]) * ni_ref[...]
        h1 = jnp.dot(t, W1_ref[...], preferred_element_type=jnp.float32)
        h1 = jnp.maximum(h1 + b1_ref[...], 0.0)
        w = jnp.sum(wp_ref[...], axis=1, keepdims=True) * no_ref[...]
        mask = lax.broadcasted_iota(jnp.int32, (np_, 1), 0) < n_real
        w = jnp.where(mask, w, 0.0)
        v = jnp.sum(h1 * w, axis=0, keepdims=True)
        readout = jnp.dot(v, W2_ref[...], preferred_element_type=jnp.float32)
        readout = readout * (1.0 / n_real) + b2_ref[...]
        out = jnp.dot(readout, Wl_ref[...], preferred_element_type=jnp.float32)
        out_ref[...] = out + bl_ref[...]

    return pl.pallas_call(
        body,
        out_shape=jax.ShapeDtypeStruct((1, bl.shape[-1]), jnp.float32),
    )(agg0, agg1, w_p, ni, no, W1, b1, W2, b2, Wl, bl)


def kernel(x, edge_index, W1, b1, W2, b2, Wl, bl):
    n, d = x.shape
    e = edge_index.shape[1]

    np_ = ((n + NS * 128 - 1) // (NS * 128)) * (NS * 128)
    pad_node = np_ - 1

    def _rup4(ne):
        return max(4, 4 * ((ne + 4 * NS * C - 1) // (4 * NS * C)))

    e0 = int(e * F0)
    nch0 = _rup4(e0)
    e0 = min(e, nch0 * C * NS)
    nch1 = _rup4(e - e0)
    nch = max(nch0, nch1)

    src = edge_index[0].astype(jnp.int32)
    dst = edge_index[1].astype(jnp.int32)

    def _layout(v):
        fill = jnp.full(((nch0 + nch1) * C * NS - e,), pad_node, jnp.int32)
        vp = jnp.concatenate([v, fill])
        v0 = vp[:nch0 * C * NS].reshape(NS, nch0, C)
        v1 = vp[nch0 * C * NS:].reshape(NS, nch1, C)
        if nch0 < nch:
            v0 = jnp.concatenate(
                [v0, jnp.full((NS, nch - nch0, C), pad_node, jnp.int32)], 1)
        if nch1 < nch:
            v1 = jnp.concatenate(
                [v1, jnp.full((NS, nch - nch1, C), pad_node, jnp.int32)], 1)
        return jnp.concatenate([v0, v1], 0)

    src3 = _layout(src)
    dst3 = _layout(dst)
    idx4 = jnp.stack([src3, dst3], axis=2)

    x_pad = jnp.zeros((np_, d), jnp.float32).at[:n].set(x)

    zeros_np = jnp.zeros((np_,), jnp.float32)
    zeros2 = jnp.zeros((ZC, d), jnp.float32)

    deg_out_p, deg_in_p = _sc_degrees(np_, nch, nch0, nch1, idx4, zeros_np,
                                      jnp.ones((16,), jnp.float32))

    xs, norm_out, norm_in = _tc_norms(deg_out_p.T, deg_in_p.T, x_pad)

    agg_p, w_p = _sc_spmm(np_, nch, nch0, nch1, xs, norm_in.reshape(np_),
                          idx4, zeros_np, zeros2)

    out = _tc_final(n, agg_p[0], agg_p[1], w_p.T,
                    norm_in, norm_out, W1, b1[None, :], W2, b2[None, :],
                    Wl, bl[None, :])
    return out
